# Initial kernel scaffold; baseline (speedup 1.0000x reference)
import jax
import jax.numpy as jnp
from jax.experimental import pallas as pl


def kernel(x, edge_index2, edge_attr2, batch, p1, p2, p3, att1W, att1b, att2W, att2b):
    raise NotImplementedError("write your pallas kernel here")


# speedup vs baseline: 12.6944x; 12.6944x over previous
import functools

import jax
import jax.numpy as jnp
from jax import lax
from jax.experimental import pallas as pl
from jax.experimental.pallas import tpu as pltpu
from jax.experimental.pallas import tpu_sc as plsc

N = 10000
E = 320000
D = 128
NE = 16
NOUT1 = 32
NOUT2 = 16
NIN = NOUT1 + NOUT2
NGRAPHS = 64
PCOLS = NE * NOUT1

NC, NS = 2, 16
NW = NC * NS
EW = E // NW
BLK = 80
NBLK = EW // BLK
RPS = 624

BE = 4000
BN = 2000


def _f32dot(a, b):
    return jnp.dot(a, b, preferred_element_type=jnp.float32)


def _edge_mlp_body(ea_ref, *refs):
    ws = refs[:9]
    outs = refs[9:]
    ea = ea_ref[...]
    aea = jnp.abs(ea)
    for li in range(3):
        w1, w2, w3 = ws[3 * li:3 * li + 3]
        t1 = jnp.maximum(_f32dot(ea, w1[...]), 0.0)
        t2 = _f32dot(aea, w2[...])
        t2 = t2 * t2
        cat = jnp.concatenate([t1, t2], axis=1)
        outs[li][...] = jnp.maximum(_f32dot(cat, w3[...]), 0.0)


def _edge_mlp(edge_attr, p1, p2, p3):
    ws = [p1["fc1_1"], p1["fc1_2"], p1["fc1_3"],
          p2["fc1_1"], p2["fc1_2"], p2["fc1_3"],
          p3["fc1_1"], p3["fc1_2"], p3["fc1_3"]]
    wspecs = [pl.BlockSpec(w.shape, lambda i: (0, 0)) for w in ws]
    return pl.pallas_call(
        _edge_mlp_body,
        grid=(E // BE,),
        in_specs=[pl.BlockSpec((BE, NE), lambda i: (i, 0))] + wspecs,
        out_specs=[pl.BlockSpec((BE, NE), lambda i: (i, 0))] * 3,
        out_shape=[jax.ShapeDtypeStruct((E, NE), jnp.float32)] * 3,
    )(edge_attr, *ws)


def _dense1_body(x_ref, wf_ref, w11_ref, b11_ref, w12_ref, b12_ref,
                 p_ref, g_ref):
    xb = x_ref[...]
    p_ref[...] = _f32dot(xb, wf_ref[...])
    g1 = jnp.maximum(_f32dot(xb, w11_ref[...]) + b11_ref[...], 0.0)
    g2 = jnp.maximum(_f32dot(xb, w12_ref[...]) + b12_ref[...], 0.0)
    g_ref[...] = g1 * g2


def _dense1(x, wf, w11, b11, w12, b12):
    full = lambda a: pl.BlockSpec(a.shape, lambda i: (0,) * a.ndim)
    return pl.pallas_call(
        _dense1_body,
        grid=(N // BN,),
        in_specs=[pl.BlockSpec((BN, D), lambda i: (i, 0)),
                  full(wf), full(w11), full(b11), full(w12), full(b12)],
        out_specs=[pl.BlockSpec((BN, PCOLS), lambda i: (i, 0)),
                   pl.BlockSpec((BN, NOUT2), lambda i: (i, 0))],
        out_shape=[jax.ShapeDtypeStruct((N, PCOLS), jnp.float32),
                   jax.ShapeDtypeStruct((N, NOUT2), jnp.float32)],
    )(x, wf, w11, b11, w12, b12)


def _dense23_body(s_ref, gp_ref, cb_ref, wf_ref, w11_ref, b11_ref,
                  w12_ref, b12_ref, p_ref, g_ref):
    conv = jnp.maximum(s_ref[0] + s_ref[1] + cb_ref[...], 0.0)
    h = jnp.concatenate([conv, gp_ref[...]], axis=1)
    p_ref[...] = _f32dot(h, wf_ref[...])
    g1 = jnp.maximum(_f32dot(h, w11_ref[...]) + b11_ref[...], 0.0)
    g2 = jnp.maximum(_f32dot(h, w12_ref[...]) + b12_ref[...], 0.0)
    g_ref[...] = g1 * g2


def _dense23(s, gp, cb, wf, w11, b11, w12, b12):
    full = lambda a: pl.BlockSpec(a.shape, lambda i: (0,) * a.ndim)
    return pl.pallas_call(
        _dense23_body,
        grid=(N // BN,),
        in_specs=[pl.BlockSpec((NC, BN, NOUT1), lambda i: (0, i, 0)),
                  pl.BlockSpec((BN, NOUT2), lambda i: (i, 0)),
                  full(cb), full(wf), full(w11), full(b11), full(w12),
                  full(b12)],
        out_specs=[pl.BlockSpec((BN, PCOLS), lambda i: (i, 0)),
                   pl.BlockSpec((BN, NOUT2), lambda i: (i, 0))],
        out_shape=[jax.ShapeDtypeStruct((N, PCOLS), jnp.float32),
                   jax.ShapeDtypeStruct((N, NOUT2), jnp.float32)],
    )(s, gp, cb, wf, w11, b11, w12, b12)


def _final_body(s_ref, gp_ref, cb_ref, batch_ref, a1w_ref, a1b_ref,
                a2w_ref, a2b_ref, out_ref, sums_ref, counts_ref):
    i = pl.program_id(0)

    @pl.when(i == 0)
    def _():
        sums_ref[...] = jnp.zeros_like(sums_ref)
        counts_ref[...] = jnp.zeros_like(counts_ref)

    conv = jnp.maximum(s_ref[0] + s_ref[1] + cb_ref[...], 0.0)
    h = jnp.concatenate([conv, gp_ref[...]], axis=1)
    bids = batch_ref[0]
    gi = lax.broadcasted_iota(jnp.int32, (NGRAPHS, BN), 0)
    oh = (bids == gi).astype(jnp.float32)
    sums_ref[...] += _f32dot(oh, h)
    counts_ref[...] += jnp.sum(oh, axis=1, keepdims=True)

    @pl.when(i == (N // BN) - 1)
    def _():
        pooled = sums_ref[...] / jnp.maximum(counts_ref[...], 1.0)
        a = jnp.maximum(_f32dot(pooled, a1w_ref[...]) + a1b_ref[...], 0.0)
        out_ref[...] = _f32dot(a, a2w_ref[...]) + a2b_ref[...]


def _final(s, gp, cb, batch3, a1w, a1b, a2w, a2b):
    full = lambda a: pl.BlockSpec(a.shape, lambda i: (0,) * a.ndim)
    return pl.pallas_call(
        _final_body,
        grid=(N // BN,),
        in_specs=[pl.BlockSpec((NC, BN, NOUT1), lambda i: (0, i, 0)),
                  pl.BlockSpec((BN, NOUT2), lambda i: (i, 0)),
                  full(cb),
                  pl.BlockSpec((1, 1, BN), lambda i: (i, 0, 0)),
                  full(a1w), full(a1b), full(a2w), full(a2b)],
        out_specs=pl.BlockSpec((NGRAPHS, 1), lambda i: (0, 0)),
        out_shape=jax.ShapeDtypeStruct((NGRAPHS, 1), jnp.float32),
        scratch_shapes=[pltpu.VMEM((NGRAPHS, NIN), jnp.float32),
                        pltpu.VMEM((NGRAPHS, 1), jnp.float32)],
    )(s, gp, cb, batch3, a1w, a1b, a2w, a2b)


def _sc_edge(p_nodes, src3, dst3, ea3):
    mesh = plsc.VectorSubcoreMesh(core_axis_name="c", subcore_axis_name="s",
                                  num_cores=NC, num_subcores=NS)
    NLAST = N - (NS - 1) * RPS

    @functools.partial(
        pl.kernel,
        out_type=jax.ShapeDtypeStruct((NC, N, NOUT1), jnp.float32),
        mesh=mesh,
        compiler_params=pltpu.CompilerParams(use_tc_tiling_on_sc=False),
        scratch_types=[
            pltpu.VMEM((BLK,), jnp.int32),
            pltpu.VMEM((BLK,), jnp.int32),
            pltpu.VMEM((BLK, NE), jnp.float32),
            pltpu.VMEM((BLK, PCOLS), jnp.float32),
            pltpu.VMEM((BLK, NOUT1), jnp.float32),
            pltpu.VMEM((NLAST, NOUT1), jnp.float32),
            pltpu.VMEM_SHARED((N, NOUT1), jnp.float32),
            pltpu.SemaphoreType.DMA,
        ],
    )
    def k(p_hbm, src_hbm, dst_hbm, ea_hbm, out_hbm,
          src_v, dst_v, ea_v, rows_v, y_v, zb_v, accum, sem):
        c = lax.axis_index("c")
        s = lax.axis_index("s")
        w = s * NC + c

        def zrow(r, carry):
            zb_v[r, pl.ds(0, 16)] = jnp.zeros((16,), jnp.float32)
            zb_v[r, pl.ds(16, 16)] = jnp.zeros((16,), jnp.float32)
            return carry

        lax.fori_loop(0, NLAST, zrow, 0)

        @pl.when(s < NS - 1)
        def _():
            pltpu.sync_copy(zb_v.at[pl.ds(0, RPS)],
                            accum.at[pl.ds(s * RPS, RPS)])

        @pl.when(s == NS - 1)
        def _():
            pltpu.sync_copy(zb_v, accum.at[pl.ds((NS - 1) * RPS, NLAST)])

        plsc.subcore_barrier()

        def body(b, carry):
            eb = w * NBLK + b
            pltpu.sync_copy(src_hbm.at[eb], src_v)
            pltpu.sync_copy(dst_hbm.at[eb], dst_v)
            pltpu.sync_copy(ea_hbm.at[eb], ea_v)
            pltpu.async_copy(p_hbm.at[src_v], rows_v, sem).wait()

            def edge(e, carry2):
                ear = ea_v[e, pl.ds(0, NE)]
                y0 = jnp.zeros((16,), jnp.float32)
                y1 = jnp.zeros((16,), jnp.float32)
                for i in range(NE):
                    sc = ear[i]
                    y0 = y0 + sc * rows_v[e, pl.ds(i * NOUT1, 16)]
                    y1 = y1 + sc * rows_v[e, pl.ds(i * NOUT1 + 16, 16)]
                y_v[e, pl.ds(0, 16)] = y0
                y_v[e, pl.ds(16, 16)] = y1
                return carry2

            lax.fori_loop(0, BLK, edge, 0)
            pltpu.sync_copy(y_v, accum.at[dst_v], add=True)
            return carry

        lax.fori_loop(0, NBLK, body, 0)
        plsc.subcore_barrier()

        @pl.when(s < NS - 1)
        def _():
            pltpu.sync_copy(accum.at[pl.ds(s * RPS, RPS)],
                            zb_v.at[pl.ds(0, RPS)])
            pltpu.sync_copy(zb_v.at[pl.ds(0, RPS)],
                            out_hbm.at[c, pl.ds(s * RPS, RPS)])

        @pl.when(s == NS - 1)
        def _():
            pltpu.sync_copy(accum.at[pl.ds((NS - 1) * RPS, NLAST)], zb_v)
            pltpu.sync_copy(zb_v, out_hbm.at[c, pl.ds((NS - 1) * RPS, NLAST)])

    return k(p_nodes, src3, dst3, ea3)


def _wflat(p):
    return jnp.transpose(p["convW"], (1, 0, 2)).reshape(-1, PCOLS)


def _row(v):
    return v.reshape(1, -1)


def kernel(x, edge_index2, edge_attr2, batch, p1, p2, p3,
           att1W, att1b, att2W, att2b):
    src = edge_index2[0].astype(jnp.int32)
    dst = edge_index2[1].astype(jnp.int32)
    ea1, ea2, ea3 = _edge_mlp(edge_attr2, p1, p2, p3)
    src3 = src.reshape(NW * NBLK, BLK)
    dst3 = dst.reshape(NW * NBLK, BLK)
    ea_r = lambda ea: ea.reshape(NW * NBLK, BLK, NE)

    P1, g1 = _dense1(x, _wflat(p1), p1["fc11W"], _row(p1["fc11b"]),
                     p1["fc12W"], _row(p1["fc12b"]))
    S1 = _sc_edge(P1, src3, dst3, ea_r(ea1))
    P2, g2 = _dense23(S1, g1, _row(p1["convb"]), _wflat(p2),
                      p2["fc11W"], _row(p2["fc11b"]),
                      p2["fc12W"], _row(p2["fc12b"]))
    S2 = _sc_edge(P2, src3, dst3, ea_r(ea2))
    P3, g3 = _dense23(S2, g2, _row(p2["convb"]), _wflat(p3),
                      p3["fc11W"], _row(p3["fc11b"]),
                      p3["fc12W"], _row(p3["fc12b"]))
    S3 = _sc_edge(P3, src3, dst3, ea_r(ea3))

    batch3 = batch.astype(jnp.int32).reshape(N // BN, 1, BN)
    return _final(S3, g3, _row(p3["convb"]), batch3,
                  att1W, _row(att1b), att2W, _row(att2b))
